# baseline (speedup 1.0000x reference)
import functools

import jax
import jax.numpy as jnp
from jax import lax
from jax.experimental import pallas as pl
from jax.experimental.pallas import tpu as pltpu
from jax.experimental.pallas import tpu_sc as plsc

NC = 2
NS = 16
NW = NC * NS
LANE = 128
KB = 8
W = 8
VL = 16

_SC_PARAMS = pltpu.CompilerParams(use_tc_tiling_on_sc=False,
                                  needs_layout_passes=False)
_MESH = dict(core_axis_name="c", subcore_axis_name="s")


def _pad_to(n, m):
  return ((n + m - 1) // m) * m


def _iota():
  return lax.iota(jnp.int32, VL)


def _splat_i(v):
  return jnp.full((VL,), v, jnp.int32)


def _rsqrt16(v):
  i = plsc.bitcast(v, jnp.int32)
  i = _splat_i(0x5F3759DF) - (i >> 1)
  y = plsc.bitcast(i, jnp.float32)
  for _ in range(3):
    y = y * (1.5 - 0.5 * v * y * y)
  return y


def _log16(v):
  y = (v - 1.0) * 0.7
  for _ in range(3):
    y = y + v * jnp.exp(-y) - 1.0
  return y


def _tile_span(tid, e_rows):
  q, r = e_rows // NW, e_rows % NW
  start = tid * q + jnp.minimum(tid, r)
  cnt = q + jnp.where(tid < r, 1, 0)
  return start, cnt


def _edge_phase(srcm_hbm, dstm_hbm, table_hbm, acc_s, src_i, dst_i, rows_v,
                sem_i, sem_g, sem_s, tid, e_rows):
  start, cnt = _tile_span(tid, e_rows)
  nfull = cnt // KB

  def body(i, _):
    roff = start + i * KB
    d1 = pltpu.async_copy(srcm_hbm.at[pl.ds(roff, KB)], src_i, sem_i)
    d2 = pltpu.async_copy(dstm_hbm.at[pl.ds(roff, KB)], dst_i, sem_i)
    d1.wait()
    d2.wait()
    gd = [pltpu.async_copy(table_hbm.at[src_i.at[j]], rows_v.at[j], sem_g)
          for j in range(KB)]
    sd = []
    for j in range(KB):
      gd[j].wait()
      sd.append(pltpu.async_copy(rows_v.at[j], acc_s.at[dst_i.at[j]],
                                 sem_s, add=True))
    for d in sd:
      d.wait()
    return 0

  lax.fori_loop(0, nfull, body, 0)

  def tail(i, _):
    pltpu.sync_copy(srcm_hbm.at[pl.ds(start + i, 1)], src_i.at[pl.ds(0, 1)])
    pltpu.sync_copy(dstm_hbm.at[pl.ds(start + i, 1)], dst_i.at[pl.ds(0, 1)])
    pltpu.async_copy(table_hbm.at[src_i.at[0]], rows_v.at[0], sem_g).wait()
    pltpu.sync_copy(rows_v.at[0], acc_s.at[dst_i.at[0]], add=True)
    return 0

  lax.fori_loop(nfull * KB, cnt, tail, 0)


def _sc_degree(dstm, zeros8, ones8, n_pad, e_rows):
  rows_per = n_pad // NS

  @functools.partial(
      pl.kernel,
      mesh=plsc.VectorSubcoreMesh(**_MESH),
      compiler_params=_SC_PARAMS,
      out_type=jax.ShapeDtypeStruct((NC, n_pad, W), jnp.float32),
      scratch_types=[
          pltpu.VMEM((KB, LANE), jnp.int32),
          pltpu.VMEM((LANE, W), jnp.float32),
          pltpu.VMEM_SHARED((n_pad, W), jnp.float32),
          pltpu.SemaphoreType.DMA,
      ],
  )
  def k(dstm_hbm, zeros_hbm, ones_hbm, out_hbm, dst_i, ones_v, acc_s, sem_s):
    c = lax.axis_index("c")
    s = lax.axis_index("s")
    tid = c * NS + s
    base = s * rows_per
    pltpu.sync_copy(zeros_hbm.at[pl.ds(base, rows_per)],
                    acc_s.at[pl.ds(base, rows_per)])
    pltpu.sync_copy(ones_hbm, ones_v)
    plsc.subcore_barrier()

    start, cnt = _tile_span(tid, e_rows)
    nfull = cnt // KB

    def body(i, _):
      roff = start + i * KB
      pltpu.sync_copy(dstm_hbm.at[pl.ds(roff, KB)], dst_i)
      sd = [pltpu.async_copy(ones_v, acc_s.at[dst_i.at[j]], sem_s, add=True)
            for j in range(KB)]
      for d in sd:
        d.wait()
      return 0

    lax.fori_loop(0, nfull, body, 0)

    def tail(i, _):
      pltpu.sync_copy(dstm_hbm.at[pl.ds(start + i, 1)], dst_i.at[pl.ds(0, 1)])
      pltpu.sync_copy(ones_v, acc_s.at[dst_i.at[0]], add=True)
      return 0

    lax.fori_loop(nfull * KB, cnt, tail, 0)
    plsc.subcore_barrier()
    pltpu.sync_copy(acc_s.at[pl.ds(base, rows_per)],
                    out_hbm.at[c, pl.ds(base, rows_per)])

  return k(dstm, zeros8, ones8)


_CH = 224


def _sc_layer1(degp, x_pad, zeros8, srcm, dstm, n_pad, e_rows):
  rows_per = n_pad // NS
  n_ch = rows_per // _CH

  @functools.partial(
      pl.kernel,
      mesh=plsc.VectorSubcoreMesh(**_MESH),
      compiler_params=_SC_PARAMS,
      out_type=(
          jax.ShapeDtypeStruct((NC, n_pad, W), jnp.float32),
          jax.ShapeDtypeStruct((n_pad, W), jnp.float32),
          jax.ShapeDtypeStruct((n_pad, 1), jnp.float32),
      ),
      scratch_types=[
          pltpu.VMEM((KB, LANE), jnp.int32),
          pltpu.VMEM((KB, LANE), jnp.int32),
          pltpu.VMEM((KB, LANE, W), jnp.float32),
          pltpu.VMEM((_CH, W), jnp.float32),
          pltpu.VMEM((_CH, W), jnp.float32),
          pltpu.VMEM((_CH, 2), jnp.float32),
          pltpu.VMEM((_CH, W), jnp.float32),
          pltpu.VMEM((_CH, 1), jnp.float32),
          pltpu.VMEM_SHARED((n_pad, W), jnp.float32),
          pltpu.SemaphoreType.DMA,
          pltpu.SemaphoreType.DMA,
          pltpu.SemaphoreType.DMA,
      ],
  )
  def k(degp_hbm, x_hbm, zeros_hbm, srcm_hbm, dstm_hbm, accp_hbm, p8_hbm, dis_hbm,
        src_i, dst_i, rows_v, d0b, d1b, xb, pb, disb,
        acc_s, sem_i, sem_g, sem_s):
    c = lax.axis_index("c")
    s = lax.axis_index("s")
    tid = c * NS + s
    base = s * rows_per
    pltpu.sync_copy(zeros_hbm.at[pl.ds(base, rows_per)],
                    acc_s.at[pl.ds(base, rows_per)])
    iota = _iota()
    zi = _splat_i(0)

    for ch in range(n_ch):
      g0 = base + ch * _CH
      pltpu.sync_copy(degp_hbm.at[0, pl.ds(g0, _CH)], d0b)
      pltpu.sync_copy(degp_hbm.at[1, pl.ds(g0, _CH)], d1b)
      pltpu.sync_copy(x_hbm.at[pl.ds(g0, _CH)], xb)
      pltpu.sync_copy(zeros_hbm.at[pl.ds(0, _CH)], pb)

      def srow(i, _):
        r = i * VL
        ridx = iota + r
        deg = (plsc.load_gather(d0b, [ridx, zi])
               + plsc.load_gather(d1b, [ridx, zi]) + 1.0)
        dis = _rsqrt16(deg)
        x0 = plsc.load_gather(xb, [ridx, zi])
        x1 = plsc.load_gather(xb, [ridx, _splat_i(1)])
        plsc.store_scatter(pb, [ridx, zi], dis * x0)
        plsc.store_scatter(pb, [ridx, _splat_i(1)], dis * x1)
        plsc.store_scatter(disb, [ridx, zi], dis)
        return 0

      lax.fori_loop(0, _CH // VL, srow, 0)
      pltpu.sync_copy(pb, p8_hbm.at[pl.ds(g0, _CH)])
      pltpu.sync_copy(disb, dis_hbm.at[pl.ds(g0, _CH)])

    plsc.subcore_barrier()
    _edge_phase(srcm_hbm, dstm_hbm, p8_hbm, acc_s, src_i, dst_i, rows_v,
                sem_i, sem_g, sem_s, tid, e_rows)
    plsc.subcore_barrier()
    pltpu.sync_copy(acc_s.at[pl.ds(base, rows_per)],
                    accp_hbm.at[c, pl.ds(base, rows_per)])

  return k(degp, x_pad, zeros8, srcm, dstm)


def _sc_layer2(accp, p8, dis, wvec, zeros8, srcm, dstm, n_pad, e_rows):
  rows_per = n_pad // NS
  n_ch = rows_per // _CH

  @functools.partial(
      pl.kernel,
      mesh=plsc.VectorSubcoreMesh(**_MESH),
      compiler_params=_SC_PARAMS,
      out_type=(
          jax.ShapeDtypeStruct((NC, n_pad, W), jnp.float32),
          jax.ShapeDtypeStruct((n_pad, W), jnp.float32),
      ),
      scratch_types=[
          pltpu.VMEM((KB, LANE), jnp.int32),
          pltpu.VMEM((KB, LANE), jnp.int32),
          pltpu.VMEM((KB, LANE, W), jnp.float32),
          pltpu.VMEM((_CH, W), jnp.float32),
          pltpu.VMEM((_CH, W), jnp.float32),
          pltpu.VMEM((_CH, W), jnp.float32),
          pltpu.VMEM((_CH, W), jnp.float32),
          pltpu.VMEM((_CH, 1), jnp.float32),
          pltpu.VMEM((32, 1), jnp.float32),
          pltpu.VMEM_SHARED((n_pad, W), jnp.float32),
          pltpu.SemaphoreType.DMA,
          pltpu.SemaphoreType.DMA,
          pltpu.SemaphoreType.DMA,
      ],
  )
  def k(accp_hbm, p8_hbm, dis_hbm, w_hbm, zeros_hbm, srcm_hbm, dstm_hbm,
        acc2_hbm, q8_hbm,
        src_i, dst_i, rows_v, a0b, a1b, pb, qb, disb, wb,
        acc_s, sem_i, sem_g, sem_s):
    c = lax.axis_index("c")
    s = lax.axis_index("s")
    tid = c * NS + s
    base = s * rows_per
    pltpu.sync_copy(zeros_hbm.at[pl.ds(base, rows_per)],
                    acc_s.at[pl.ds(base, rows_per)])
    pltpu.sync_copy(w_hbm, wb)
    iota = _iota()
    zi = _splat_i(0)

    def wsp(i):
      return plsc.load_gather(wb, [_splat_i(i), zi])

    w1 = [[wsp(kk * 4 + j) for j in range(4)] for kk in range(2)]
    b1 = [wsp(8 + j) for j in range(4)]
    w2 = [[wsp(12 + j * 2 + col) for col in range(2)] for j in range(4)]

    for ch in range(n_ch):
      g0 = base + ch * _CH
      pltpu.sync_copy(accp_hbm.at[0, pl.ds(g0, _CH)], a0b)
      pltpu.sync_copy(accp_hbm.at[1, pl.ds(g0, _CH)], a1b)
      pltpu.sync_copy(p8_hbm.at[pl.ds(g0, _CH)], pb)
      pltpu.sync_copy(dis_hbm.at[pl.ds(g0, _CH)], disb)
      pltpu.sync_copy(zeros_hbm.at[pl.ds(0, _CH)], qb)

      def srow(i, _):
        r = i * VL
        ridx = iota + r
        oi = _splat_i(1)
        dis = plsc.load_gather(disb, [ridx, zi])
        t0 = dis * (plsc.load_gather(a0b, [ridx, zi])
                    + plsc.load_gather(a1b, [ridx, zi])
                    + plsc.load_gather(pb, [ridx, zi]))
        t1 = dis * (plsc.load_gather(a0b, [ridx, oi])
                    + plsc.load_gather(a1b, [ridx, oi])
                    + plsc.load_gather(pb, [ridx, oi]))
        q0 = jnp.zeros((VL,), jnp.float32)
        q1 = jnp.zeros((VL,), jnp.float32)
        for j in range(4):
          h = jnp.maximum(t0 * w1[0][j] + t1 * w1[1][j] + b1[j], 0.0)
          q0 = q0 + h * w2[j][0]
          q1 = q1 + h * w2[j][1]
        plsc.store_scatter(qb, [ridx, zi], dis * q0)
        plsc.store_scatter(qb, [ridx, oi], dis * q1)
        return 0

      lax.fori_loop(0, _CH // VL, srow, 0)
      pltpu.sync_copy(qb, q8_hbm.at[pl.ds(g0, _CH)])

    plsc.subcore_barrier()
    _edge_phase(srcm_hbm, dstm_hbm, q8_hbm, acc_s, src_i, dst_i, rows_v,
                sem_i, sem_g, sem_s, tid, e_rows)
    plsc.subcore_barrier()
    pltpu.sync_copy(acc_s.at[pl.ds(base, rows_per)],
                    acc2_hbm.at[c, pl.ds(base, rows_per)])

  return k(accp, p8, dis, wvec, zeros8, srcm, dstm)


def _sc_final(acc2, q8, dis, wvec, n_pad):
  rows_per = n_pad // NS
  n_ch = rows_per // _CH

  @functools.partial(
      pl.kernel,
      mesh=plsc.VectorSubcoreMesh(**_MESH),
      compiler_params=_SC_PARAMS,
      out_type=jax.ShapeDtypeStruct((n_pad, 2), jnp.float32),
      scratch_types=[
          pltpu.VMEM((_CH, W), jnp.float32),
          pltpu.VMEM((_CH, W), jnp.float32),
          pltpu.VMEM((_CH, W), jnp.float32),
          pltpu.VMEM((_CH, 1), jnp.float32),
          pltpu.VMEM((_CH, 2), jnp.float32),
          pltpu.VMEM((32, 1), jnp.float32),
      ],
  )
  def k(acc2_hbm, q8_hbm, dis_hbm, w_hbm, out_hbm,
        a0b, a1b, qb, disb, ob, wb):
    c = lax.axis_index("c")
    s = lax.axis_index("s")
    base = s * rows_per
    iota = _iota()
    zi = _splat_i(0)

    @pl.when(c == 0)
    def _():
      pltpu.sync_copy(w_hbm, wb)
      b2_0 = plsc.load_gather(wb, [_splat_i(20), zi])
      b2_1 = plsc.load_gather(wb, [_splat_i(21), zi])
      for ch in range(n_ch):
        g0 = base + ch * _CH
        pltpu.sync_copy(acc2_hbm.at[0, pl.ds(g0, _CH)], a0b)
        pltpu.sync_copy(acc2_hbm.at[1, pl.ds(g0, _CH)], a1b)
        pltpu.sync_copy(q8_hbm.at[pl.ds(g0, _CH)], qb)
        pltpu.sync_copy(dis_hbm.at[pl.ds(g0, _CH)], disb)

        def srow(i, _):
          r = i * VL
          ridx = iota + r
          oi = _splat_i(1)
          dis = plsc.load_gather(disb, [ridx, zi])
          o0 = dis * (plsc.load_gather(a0b, [ridx, zi])
                      + plsc.load_gather(a1b, [ridx, zi])
                      + plsc.load_gather(qb, [ridx, zi])) + b2_0
          o1 = dis * (plsc.load_gather(a0b, [ridx, oi])
                      + plsc.load_gather(a1b, [ridx, oi])
                      + plsc.load_gather(qb, [ridx, oi])) + b2_1
          m = jnp.maximum(o0, o1)
          s0 = o0 - m
          s1 = o1 - m
          ls = _log16(jnp.exp(s0) + jnp.exp(s1))
          plsc.store_scatter(ob, [ridx, zi], s0 - ls)
          plsc.store_scatter(ob, [ridx, oi], s1 - ls)
          return 0

        lax.fori_loop(0, _CH // VL, srow, 0)
        pltpu.sync_copy(ob, out_hbm.at[pl.ds(g0, _CH)])

  return k(acc2, q8, dis, wvec)


def kernel(x, edge_index, W1, b1, W2, b2):
  n = x.shape[0]
  e = edge_index.shape[1]
  n_pad = _pad_to(n, NS * _CH)

  if e % LANE == 0:
    ei = edge_index
  else:
    if n_pad == n:
      n_pad += NS * _CH
    e_pad = _pad_to(e, LANE)
    fill = jnp.full((2, e_pad - e), n, jnp.int32)
    ei = jnp.concatenate([edge_index, fill], axis=1)
  e_rows = ei.shape[1] // LANE
  srcm = ei[0].reshape(e_rows, LANE)
  dstm = ei[1].reshape(e_rows, LANE)

  x_pad = jnp.zeros((n_pad, 2), jnp.float32).at[:n].set(x)
  zeros8 = jnp.zeros((n_pad, W), jnp.float32)
  ones8 = jnp.zeros((LANE, W), jnp.float32).at[:, 0].set(1.0)
  wvec = jnp.concatenate([
      W1.reshape(-1), b1.reshape(-1), W2.reshape(-1), b2.reshape(-1),
      jnp.zeros((10,), jnp.float32)]).reshape(32, 1)

  degp = _sc_degree(dstm, zeros8, ones8, n_pad, e_rows)
  accp, p8, dis = _sc_layer1(degp, x_pad, zeros8, srcm, dstm, n_pad, e_rows)
  acc2, q8 = _sc_layer2(accp, p8, dis, wvec, zeros8, srcm, dstm, n_pad, e_rows)
  out = _sc_final(acc2, q8, dis, wvec, n_pad)
  return out[:n]


# speedup vs baseline: 1.3245x; 1.3245x over previous
import functools

import jax
import jax.numpy as jnp
from jax import lax
from jax.experimental import pallas as pl
from jax.experimental.pallas import tpu as pltpu
from jax.experimental.pallas import tpu_sc as plsc

NC = 2
NS = 16
NW = NC * NS
LANE = 128
KB = 16
W = 8
VL = 16

_SC_PARAMS = pltpu.CompilerParams(use_tc_tiling_on_sc=False,
                                  needs_layout_passes=False)
_MESH = dict(core_axis_name="c", subcore_axis_name="s")


def _pad_to(n, m):
  return ((n + m - 1) // m) * m


def _iota():
  return lax.iota(jnp.int32, VL)


def _splat_i(v):
  return jnp.full((VL,), v, jnp.int32)


def _rsqrt16(v):
  i = plsc.bitcast(v, jnp.int32)
  i = _splat_i(0x5F3759DF) - (i >> 1)
  y = plsc.bitcast(i, jnp.float32)
  for _ in range(3):
    y = y * (1.5 - 0.5 * v * y * y)
  return y


def _log16(v):
  y = (v - 1.0) * 0.7
  for _ in range(3):
    y = y + v * jnp.exp(-y) - 1.0
  return y


def _tile_span(tid, e_rows):
  q, r = e_rows // NW, e_rows % NW
  start = tid * q + jnp.minimum(tid, r)
  cnt = q + jnp.where(tid < r, 1, 0)
  return start, cnt


def _edge_phase(srcm_hbm, dstm_hbm, table_hbm, acc_s, src_i, dst_i, rows_v,
                sem_i, sem_g, sem_s, tid, e_rows):
  start, cnt = _tile_span(tid, e_rows)
  nfull = cnt // KB

  def body(i, _):
    roff = start + i * KB
    d1 = pltpu.async_copy(srcm_hbm.at[pl.ds(roff, KB)], src_i, sem_i)
    d2 = pltpu.async_copy(dstm_hbm.at[pl.ds(roff, KB)], dst_i, sem_i)
    d1.wait()
    d2.wait()
    gd = [pltpu.async_copy(table_hbm.at[src_i.at[j]], rows_v.at[j], sem_g)
          for j in range(KB)]
    sd = []
    for j in range(KB):
      gd[j].wait()
      sd.append(pltpu.async_copy(rows_v.at[j], acc_s.at[dst_i.at[j]],
                                 sem_s, add=True))
    for d in sd:
      d.wait()
    return 0

  lax.fori_loop(0, nfull, body, 0)

  def tail(i, _):
    pltpu.sync_copy(srcm_hbm.at[pl.ds(start + i, 1)], src_i.at[pl.ds(0, 1)])
    pltpu.sync_copy(dstm_hbm.at[pl.ds(start + i, 1)], dst_i.at[pl.ds(0, 1)])
    pltpu.async_copy(table_hbm.at[src_i.at[0]], rows_v.at[0], sem_g).wait()
    pltpu.sync_copy(rows_v.at[0], acc_s.at[dst_i.at[0]], add=True)
    return 0

  lax.fori_loop(nfull * KB, cnt, tail, 0)


def _sc_degree(dstm, zeros8, ones8, n_pad, e_rows):
  rows_per = n_pad // NS

  @functools.partial(
      pl.kernel,
      mesh=plsc.VectorSubcoreMesh(**_MESH),
      compiler_params=_SC_PARAMS,
      out_type=jax.ShapeDtypeStruct((NC, n_pad, W), jnp.float32),
      scratch_types=[
          pltpu.VMEM((KB, LANE), jnp.int32),
          pltpu.VMEM((LANE, W), jnp.float32),
          pltpu.VMEM_SHARED((n_pad, W), jnp.float32),
          pltpu.SemaphoreType.DMA,
      ],
  )
  def k(dstm_hbm, zeros_hbm, ones_hbm, out_hbm, dst_i, ones_v, acc_s, sem_s):
    c = lax.axis_index("c")
    s = lax.axis_index("s")
    tid = c * NS + s
    base = s * rows_per
    pltpu.sync_copy(zeros_hbm.at[pl.ds(base, rows_per)],
                    acc_s.at[pl.ds(base, rows_per)])
    pltpu.sync_copy(ones_hbm, ones_v)
    plsc.subcore_barrier()

    start, cnt = _tile_span(tid, e_rows)
    nfull = cnt // KB

    def body(i, _):
      roff = start + i * KB
      pltpu.sync_copy(dstm_hbm.at[pl.ds(roff, KB)], dst_i)
      sd = [pltpu.async_copy(ones_v, acc_s.at[dst_i.at[j]], sem_s, add=True)
            for j in range(KB)]
      for d in sd:
        d.wait()
      return 0

    lax.fori_loop(0, nfull, body, 0)

    def tail(i, _):
      pltpu.sync_copy(dstm_hbm.at[pl.ds(start + i, 1)], dst_i.at[pl.ds(0, 1)])
      pltpu.sync_copy(ones_v, acc_s.at[dst_i.at[0]], add=True)
      return 0

    lax.fori_loop(nfull * KB, cnt, tail, 0)
    plsc.subcore_barrier()
    pltpu.sync_copy(acc_s.at[pl.ds(base, rows_per)],
                    out_hbm.at[c, pl.ds(base, rows_per)])

  return k(dstm, zeros8, ones8)


_CH = 224


def _sc_layer1(degp, x_pad, zeros8, srcm, dstm, n_pad, e_rows):
  rows_per = n_pad // NS
  n_ch = rows_per // _CH

  @functools.partial(
      pl.kernel,
      mesh=plsc.VectorSubcoreMesh(**_MESH),
      compiler_params=_SC_PARAMS,
      out_type=(
          jax.ShapeDtypeStruct((NC, n_pad, W), jnp.float32),
          jax.ShapeDtypeStruct((n_pad, W), jnp.float32),
          jax.ShapeDtypeStruct((n_pad, 1), jnp.float32),
      ),
      scratch_types=[
          pltpu.VMEM((KB, LANE), jnp.int32),
          pltpu.VMEM((KB, LANE), jnp.int32),
          pltpu.VMEM((KB, LANE, W), jnp.float32),
          pltpu.VMEM((_CH, W), jnp.float32),
          pltpu.VMEM((_CH, W), jnp.float32),
          pltpu.VMEM((_CH, 2), jnp.float32),
          pltpu.VMEM((_CH, W), jnp.float32),
          pltpu.VMEM((_CH, 1), jnp.float32),
          pltpu.VMEM_SHARED((n_pad, W), jnp.float32),
          pltpu.VMEM_SHARED((n_pad, W), jnp.float32),
          pltpu.SemaphoreType.DMA,
          pltpu.SemaphoreType.DMA,
          pltpu.SemaphoreType.DMA,
      ],
  )
  def k(degp_hbm, x_hbm, zeros_hbm, srcm_hbm, dstm_hbm, accp_hbm, p8_hbm, dis_hbm,
        src_i, dst_i, rows_v, d0b, d1b, xb, pb, disb,
        table_s, acc_s, sem_i, sem_g, sem_s):
    c = lax.axis_index("c")
    s = lax.axis_index("s")
    tid = c * NS + s
    base = s * rows_per
    pltpu.sync_copy(zeros_hbm.at[pl.ds(base, rows_per)],
                    acc_s.at[pl.ds(base, rows_per)])
    iota = _iota()
    zi = _splat_i(0)

    for ch in range(n_ch):
      g0 = base + ch * _CH
      pltpu.sync_copy(degp_hbm.at[0, pl.ds(g0, _CH)], d0b)
      pltpu.sync_copy(degp_hbm.at[1, pl.ds(g0, _CH)], d1b)
      pltpu.sync_copy(x_hbm.at[pl.ds(g0, _CH)], xb)
      pltpu.sync_copy(zeros_hbm.at[pl.ds(0, _CH)], pb)

      def srow(i, _):
        r = i * VL
        ridx = iota + r
        deg = (plsc.load_gather(d0b, [ridx, zi])
               + plsc.load_gather(d1b, [ridx, zi]) + 1.0)
        dis = _rsqrt16(deg)
        x0 = plsc.load_gather(xb, [ridx, zi])
        x1 = plsc.load_gather(xb, [ridx, _splat_i(1)])
        plsc.store_scatter(pb, [ridx, zi], dis * x0)
        plsc.store_scatter(pb, [ridx, _splat_i(1)], dis * x1)
        plsc.store_scatter(disb, [ridx, zi], dis)
        return 0

      lax.fori_loop(0, _CH // VL, srow, 0)
      pltpu.sync_copy(pb, table_s.at[pl.ds(g0, _CH)])
      pltpu.sync_copy(pb, p8_hbm.at[pl.ds(g0, _CH)])
      pltpu.sync_copy(disb, dis_hbm.at[pl.ds(g0, _CH)])

    plsc.subcore_barrier()
    _edge_phase(srcm_hbm, dstm_hbm, table_s, acc_s, src_i, dst_i, rows_v,
                sem_i, sem_g, sem_s, tid, e_rows)
    plsc.subcore_barrier()
    pltpu.sync_copy(acc_s.at[pl.ds(base, rows_per)],
                    accp_hbm.at[c, pl.ds(base, rows_per)])

  return k(degp, x_pad, zeros8, srcm, dstm)


def _sc_layer2(accp, p8, dis, wvec, zeros8, srcm, dstm, n_pad, e_rows):
  rows_per = n_pad // NS
  n_ch = rows_per // _CH

  @functools.partial(
      pl.kernel,
      mesh=plsc.VectorSubcoreMesh(**_MESH),
      compiler_params=_SC_PARAMS,
      out_type=(
          jax.ShapeDtypeStruct((NC, n_pad, W), jnp.float32),
          jax.ShapeDtypeStruct((n_pad, W), jnp.float32),
      ),
      scratch_types=[
          pltpu.VMEM((KB, LANE), jnp.int32),
          pltpu.VMEM((KB, LANE), jnp.int32),
          pltpu.VMEM((KB, LANE, W), jnp.float32),
          pltpu.VMEM((_CH, W), jnp.float32),
          pltpu.VMEM((_CH, W), jnp.float32),
          pltpu.VMEM((_CH, W), jnp.float32),
          pltpu.VMEM((_CH, W), jnp.float32),
          pltpu.VMEM((_CH, 1), jnp.float32),
          pltpu.VMEM((32, 1), jnp.float32),
          pltpu.VMEM_SHARED((n_pad, W), jnp.float32),
          pltpu.VMEM_SHARED((n_pad, W), jnp.float32),
          pltpu.SemaphoreType.DMA,
          pltpu.SemaphoreType.DMA,
          pltpu.SemaphoreType.DMA,
      ],
  )
  def k(accp_hbm, p8_hbm, dis_hbm, w_hbm, zeros_hbm, srcm_hbm, dstm_hbm,
        acc2_hbm, q8_hbm,
        src_i, dst_i, rows_v, a0b, a1b, pb, qb, disb, wb,
        table_s, acc_s, sem_i, sem_g, sem_s):
    c = lax.axis_index("c")
    s = lax.axis_index("s")
    tid = c * NS + s
    base = s * rows_per
    pltpu.sync_copy(zeros_hbm.at[pl.ds(base, rows_per)],
                    acc_s.at[pl.ds(base, rows_per)])
    pltpu.sync_copy(w_hbm, wb)
    iota = _iota()
    zi = _splat_i(0)

    def wsp(i):
      return plsc.load_gather(wb, [_splat_i(i), zi])

    w1 = [[wsp(kk * 4 + j) for j in range(4)] for kk in range(2)]
    b1 = [wsp(8 + j) for j in range(4)]
    w2 = [[wsp(12 + j * 2 + col) for col in range(2)] for j in range(4)]

    for ch in range(n_ch):
      g0 = base + ch * _CH
      pltpu.sync_copy(accp_hbm.at[0, pl.ds(g0, _CH)], a0b)
      pltpu.sync_copy(accp_hbm.at[1, pl.ds(g0, _CH)], a1b)
      pltpu.sync_copy(p8_hbm.at[pl.ds(g0, _CH)], pb)
      pltpu.sync_copy(dis_hbm.at[pl.ds(g0, _CH)], disb)
      pltpu.sync_copy(zeros_hbm.at[pl.ds(0, _CH)], qb)

      def srow(i, _):
        r = i * VL
        ridx = iota + r
        oi = _splat_i(1)
        dis = plsc.load_gather(disb, [ridx, zi])
        t0 = dis * (plsc.load_gather(a0b, [ridx, zi])
                    + plsc.load_gather(a1b, [ridx, zi])
                    + plsc.load_gather(pb, [ridx, zi]))
        t1 = dis * (plsc.load_gather(a0b, [ridx, oi])
                    + plsc.load_gather(a1b, [ridx, oi])
                    + plsc.load_gather(pb, [ridx, oi]))
        q0 = jnp.zeros((VL,), jnp.float32)
        q1 = jnp.zeros((VL,), jnp.float32)
        for j in range(4):
          h = jnp.maximum(t0 * w1[0][j] + t1 * w1[1][j] + b1[j], 0.0)
          q0 = q0 + h * w2[j][0]
          q1 = q1 + h * w2[j][1]
        plsc.store_scatter(qb, [ridx, zi], dis * q0)
        plsc.store_scatter(qb, [ridx, oi], dis * q1)
        return 0

      lax.fori_loop(0, _CH // VL, srow, 0)
      pltpu.sync_copy(qb, table_s.at[pl.ds(g0, _CH)])
      pltpu.sync_copy(qb, q8_hbm.at[pl.ds(g0, _CH)])

    plsc.subcore_barrier()
    _edge_phase(srcm_hbm, dstm_hbm, table_s, acc_s, src_i, dst_i, rows_v,
                sem_i, sem_g, sem_s, tid, e_rows)
    plsc.subcore_barrier()
    pltpu.sync_copy(acc_s.at[pl.ds(base, rows_per)],
                    acc2_hbm.at[c, pl.ds(base, rows_per)])

  return k(accp, p8, dis, wvec, zeros8, srcm, dstm)


def _sc_final(acc2, q8, dis, wvec, n_pad):
  rows_per = n_pad // NS
  n_ch = rows_per // _CH

  @functools.partial(
      pl.kernel,
      mesh=plsc.VectorSubcoreMesh(**_MESH),
      compiler_params=_SC_PARAMS,
      out_type=jax.ShapeDtypeStruct((n_pad, 2), jnp.float32),
      scratch_types=[
          pltpu.VMEM((_CH, W), jnp.float32),
          pltpu.VMEM((_CH, W), jnp.float32),
          pltpu.VMEM((_CH, W), jnp.float32),
          pltpu.VMEM((_CH, 1), jnp.float32),
          pltpu.VMEM((_CH, 2), jnp.float32),
          pltpu.VMEM((32, 1), jnp.float32),
      ],
  )
  def k(acc2_hbm, q8_hbm, dis_hbm, w_hbm, out_hbm,
        a0b, a1b, qb, disb, ob, wb):
    c = lax.axis_index("c")
    s = lax.axis_index("s")
    base = s * rows_per
    iota = _iota()
    zi = _splat_i(0)

    @pl.when(c == 0)
    def _():
      pltpu.sync_copy(w_hbm, wb)
      b2_0 = plsc.load_gather(wb, [_splat_i(20), zi])
      b2_1 = plsc.load_gather(wb, [_splat_i(21), zi])
      for ch in range(n_ch):
        g0 = base + ch * _CH
        pltpu.sync_copy(acc2_hbm.at[0, pl.ds(g0, _CH)], a0b)
        pltpu.sync_copy(acc2_hbm.at[1, pl.ds(g0, _CH)], a1b)
        pltpu.sync_copy(q8_hbm.at[pl.ds(g0, _CH)], qb)
        pltpu.sync_copy(dis_hbm.at[pl.ds(g0, _CH)], disb)

        def srow(i, _):
          r = i * VL
          ridx = iota + r
          oi = _splat_i(1)
          dis = plsc.load_gather(disb, [ridx, zi])
          o0 = dis * (plsc.load_gather(a0b, [ridx, zi])
                      + plsc.load_gather(a1b, [ridx, zi])
                      + plsc.load_gather(qb, [ridx, zi])) + b2_0
          o1 = dis * (plsc.load_gather(a0b, [ridx, oi])
                      + plsc.load_gather(a1b, [ridx, oi])
                      + plsc.load_gather(qb, [ridx, oi])) + b2_1
          m = jnp.maximum(o0, o1)
          s0 = o0 - m
          s1 = o1 - m
          ls = _log16(jnp.exp(s0) + jnp.exp(s1))
          plsc.store_scatter(ob, [ridx, zi], s0 - ls)
          plsc.store_scatter(ob, [ridx, oi], s1 - ls)
          return 0

        lax.fori_loop(0, _CH // VL, srow, 0)
        pltpu.sync_copy(ob, out_hbm.at[pl.ds(g0, _CH)])

  return k(acc2, q8, dis, wvec)


def kernel(x, edge_index, W1, b1, W2, b2):
  n = x.shape[0]
  e = edge_index.shape[1]
  n_pad = _pad_to(n, NS * _CH)

  if e % LANE == 0:
    ei = edge_index
  else:
    if n_pad == n:
      n_pad += NS * _CH
    e_pad = _pad_to(e, LANE)
    fill = jnp.full((2, e_pad - e), n, jnp.int32)
    ei = jnp.concatenate([edge_index, fill], axis=1)
  e_rows = ei.shape[1] // LANE
  srcm = ei[0].reshape(e_rows, LANE)
  dstm = ei[1].reshape(e_rows, LANE)

  x_pad = jnp.zeros((n_pad, 2), jnp.float32).at[:n].set(x)
  zeros8 = jnp.zeros((n_pad, W), jnp.float32)
  ones8 = jnp.zeros((LANE, W), jnp.float32).at[:, 0].set(1.0)
  wvec = jnp.concatenate([
      W1.reshape(-1), b1.reshape(-1), W2.reshape(-1), b2.reshape(-1),
      jnp.zeros((10,), jnp.float32)]).reshape(32, 1)

  degp = _sc_degree(dstm, zeros8, ones8, n_pad, e_rows)
  accp, p8, dis = _sc_layer1(degp, x_pad, zeros8, srcm, dstm, n_pad, e_rows)
  acc2, q8 = _sc_layer2(accp, p8, dis, wvec, zeros8, srcm, dstm, n_pad, e_rows)
  out = _sc_final(acc2, q8, dis, wvec, n_pad)
  return out[:n]
